# Initial kernel scaffold; baseline (speedup 1.0000x reference)
import jax
import jax.numpy as jnp
from jax.experimental import pallas as pl


def kernel(center_ids, pos_ids, neg_ids, center_table, context_table):
    raise NotImplementedError("write your pallas kernel here")


# speedup vs baseline: 4.2092x; 4.2092x over previous
import jax
import jax.numpy as jnp
from jax import lax
from jax.experimental import pallas as pl
from jax.experimental.pallas import tpu as pltpu
from jax.experimental.pallas import tpu_sc as plsc

D = 64
B = 16384
K = 20
NC, NS = 2, 16
NW = NC * NS
BPW = B // NW
CH = 512
NNEG = B * K // NW
NCH = NNEG // CH


def _gather_body(cids, pids, nids, ctab, xtab, v_out, p_out, n_out,
                 idx_v, rows_v, sem):
    wid = lax.axis_index("s") * NC + lax.axis_index("c")
    base = wid * BPW
    pltpu.sync_copy(cids.at[pl.ds(base, CH)], idx_v)
    pltpu.async_copy(ctab.at[idx_v], rows_v, sem).wait()
    pltpu.sync_copy(rows_v, v_out.at[pl.ds(base, CH)])
    pltpu.sync_copy(pids.at[pl.ds(base, CH)], idx_v)
    pltpu.async_copy(xtab.at[idx_v], rows_v, sem).wait()
    pltpu.sync_copy(rows_v, p_out.at[pl.ds(base, CH)])
    nbase = wid * NNEG
    for j in range(NCH):
        pltpu.sync_copy(nids.at[pl.ds(nbase + j * CH, CH)], idx_v)
        pltpu.async_copy(xtab.at[idx_v], rows_v, sem).wait()
        pltpu.sync_copy(rows_v, n_out.at[pl.ds(nbase + j * CH, CH)])


_gather_cache = []


def _gather_kernel():
    if not _gather_cache:
        _gather_cache.append(pl.kernel(
            _gather_body,
            out_type=(
                jax.ShapeDtypeStruct((B, D), jnp.float32),
                jax.ShapeDtypeStruct((B, D), jnp.float32),
                jax.ShapeDtypeStruct((B * K, D), jnp.float32),
            ),
            mesh=plsc.VectorSubcoreMesh(
                core_axis_name="c", subcore_axis_name="s",
                num_cores=NC, num_subcores=NS),
            scratch_types=[
                pltpu.VMEM((CH,), jnp.int32),
                pltpu.VMEM((CH, D), jnp.float32),
                pltpu.SemaphoreType.DMA,
            ],
            compiler_params=pltpu.CompilerParams(use_tc_tiling_on_sc=False),
        ))
    return _gather_cache[0]

BB = 1024


def _logsig(x):
    return jnp.minimum(x, 0.0) - jnp.log1p(jnp.exp(-jnp.abs(x)))


def _score_body(v_ref, p_ref, n_ref, o_ref):
    i = pl.program_id(0)
    v = v_ref[...]
    p = p_ref[...]
    n = n_ref[...].reshape(BB, K, D)
    pos = jnp.sum(v * p, axis=1, keepdims=True)
    neg = jnp.sum(n * v[:, None, :], axis=2)
    lv = -_logsig(pos) - jnp.sum(_logsig(-neg), axis=1, keepdims=True)

    @pl.when(i == 0)
    def _():
        o_ref[...] = jnp.zeros((1, 1), jnp.float32)

    o_ref[...] += jnp.sum(lv).reshape(1, 1)


_score = pl.pallas_call(
    _score_body,
    grid=(B // BB,),
    in_specs=[
        pl.BlockSpec((BB, D), lambda i: (i, 0)),
        pl.BlockSpec((BB, D), lambda i: (i, 0)),
        pl.BlockSpec((BB * K, D), lambda i: (i, 0)),
    ],
    out_specs=pl.BlockSpec((1, 1), lambda i: (0, 0)),
    out_shape=jax.ShapeDtypeStruct((1, 1), jnp.float32),
)


def kernel(center_ids, pos_ids, neg_ids, center_table, context_table):
    cids = center_ids.astype(jnp.int32)
    pids = pos_ids.astype(jnp.int32)
    nids = neg_ids.reshape(-1).astype(jnp.int32)
    v, p, n = _gather_kernel()(cids, pids, nids, center_table, context_table)
    total = _score(v, p, n)
    return total[0, 0] / B


# speedup vs baseline: 5.1780x; 1.2302x over previous
import jax
import jax.numpy as jnp
from jax import lax
from jax.experimental import pallas as pl
from jax.experimental.pallas import tpu as pltpu
from jax.experimental.pallas import tpu_sc as plsc

D = 64
B = 16384
K = 20
SLOTS = 32
NC, NS = 2, 16
NW = NC * NS
BPW = B // NW
CHB = 64
NCHK = BPW // CHB
CN = CHB * K


def _fused_body(cids, pids, nids, ctab, xtab, s_out,
                vidx, pidx, nidx0, nidx1, nidx2,
                vrows, prows, nrows, stage, sem):
    wid = lax.axis_index("s") * NC + lax.axis_index("c")
    lanes = lax.iota(jnp.int32, 16)

    def chunk(c, carry):
        b0 = wid * BPW + c * CHB
        r0 = b0 * K
        pltpu.sync_copy(cids.at[pl.ds(b0, CHB)], vidx)
        pltpu.sync_copy(pids.at[pl.ds(b0, CHB)], pidx)
        pltpu.sync_copy(nids.at[pl.ds(r0, 512)], nidx0)
        pltpu.sync_copy(nids.at[pl.ds(r0 + 512, 512)], nidx1)
        pltpu.sync_copy(nids.at[pl.ds(r0 + 1024, 256)], nidx2)
        c1 = pltpu.async_copy(ctab.at[vidx], vrows, sem)
        c2 = pltpu.async_copy(xtab.at[pidx], prows, sem)
        c3 = pltpu.async_copy(xtab.at[nidx0], nrows.at[pl.ds(0, 512)], sem)
        c4 = pltpu.async_copy(xtab.at[nidx1], nrows.at[pl.ds(512, 512)], sem)
        c5 = pltpu.async_copy(xtab.at[nidx2], nrows.at[pl.ds(1024, 256)], sem)
        c1.wait()
        c2.wait()
        c3.wait()
        c4.wait()
        c5.wait()

        def per_b(b, carry_b):
            v0 = vrows[b, pl.ds(0, 16)]
            v1 = vrows[b, pl.ds(16, 16)]
            v2 = vrows[b, pl.ds(32, 16)]
            v3 = vrows[b, pl.ds(48, 16)]
            acc_a = jnp.zeros((16,), jnp.float32)
            acc_b = jnp.zeros((16,), jnp.float32)
            for k in range(K):
                r = b * K + k
                t = (v0 * nrows[r, pl.ds(0, 16)]
                     + v1 * nrows[r, pl.ds(16, 16)]
                     + v2 * nrows[r, pl.ds(32, 16)]
                     + v3 * nrows[r, pl.ds(48, 16)])
                s = jnp.sum(t)
                if k < 16:
                    acc_a = jnp.where(lanes == k, s, acc_a)
                else:
                    acc_b = jnp.where(lanes == (k - 16), s, acc_b)
            t = (v0 * prows[b, pl.ds(0, 16)] + v1 * prows[b, pl.ds(16, 16)]
                 + v2 * prows[b, pl.ds(32, 16)] + v3 * prows[b, pl.ds(48, 16)])
            acc_b = jnp.where(lanes == (K - 16), jnp.sum(t), acc_b)
            stage[pl.ds(b * SLOTS, 16)] = acc_a
            stage[pl.ds(b * SLOTS + 16, 16)] = acc_b
            return carry_b

        lax.fori_loop(0, CHB, per_b, 0)
        pltpu.sync_copy(stage, s_out.at[pl.ds(b0 * SLOTS, CHB * SLOTS)])
        return carry

    lax.fori_loop(0, NCHK, chunk, 0)


_fused_cache = []


def _fused_kernel():
    if not _fused_cache:
        _fused_cache.append(pl.kernel(
            _fused_body,
            out_type=jax.ShapeDtypeStruct((B * SLOTS,), jnp.float32),
            mesh=plsc.VectorSubcoreMesh(
                core_axis_name="c", subcore_axis_name="s",
                num_cores=NC, num_subcores=NS),
            scratch_types=[
                pltpu.VMEM((CHB,), jnp.int32),
                pltpu.VMEM((CHB,), jnp.int32),
                pltpu.VMEM((512,), jnp.int32),
                pltpu.VMEM((512,), jnp.int32),
                pltpu.VMEM((256,), jnp.int32),
                pltpu.VMEM((CHB, D), jnp.float32),
                pltpu.VMEM((CHB, D), jnp.float32),
                pltpu.VMEM((CN, D), jnp.float32),
                pltpu.VMEM((CHB * SLOTS,), jnp.float32),
                pltpu.SemaphoreType.DMA,
            ],
            compiler_params=pltpu.CompilerParams(
                use_tc_tiling_on_sc=False, needs_layout_passes=False),
        ))
    return _fused_cache[0]


def _logsig(x):
    return jnp.minimum(x, 0.0) - jnp.log1p(jnp.exp(-jnp.abs(x)))


def _loss_body(s_ref, o_ref):
    s = s_ref[...]
    col = lax.broadcasted_iota(jnp.int32, (B, SLOTS), 1)
    neg = jnp.where(col < K, -_logsig(-s), 0.0)
    pos = jnp.where(col == K, -_logsig(s), 0.0)
    o_ref[...] = jnp.sum(neg + pos).reshape(1, 1)


_loss = pl.pallas_call(
    _loss_body,
    out_shape=jax.ShapeDtypeStruct((1, 1), jnp.float32),
)


def kernel(center_ids, pos_ids, neg_ids, center_table, context_table):
    cids = center_ids.astype(jnp.int32)
    pids = pos_ids.astype(jnp.int32)
    nids = neg_ids.reshape(-1).astype(jnp.int32)
    scores = _fused_kernel()(cids, pids, nids, center_table, context_table)
    total = _loss(scores.reshape(B, SLOTS))
    return total[0, 0] / B


# speedup vs baseline: 5.2790x; 1.0195x over previous
import jax
import jax.numpy as jnp
from jax import lax
from jax.experimental import pallas as pl
from jax.experimental.pallas import tpu as pltpu
from jax.experimental.pallas import tpu_sc as plsc

D = 64
DP = 128
B = 16384
K = 20
SLOTS = 32
NC, NS = 2, 16
NW = NC * NS
BPW = B // NW
CHB = 32
NCHK = BPW // CHB
CN = CHB * K


def _fused_body(cids, pids, nids, ctab, xtab, s_out,
                vidx, pidx, nidx0, nidx1,
                vrows, prows, nrows, stage, sem):
    wid = lax.axis_index("s") * NC + lax.axis_index("c")
    lanes = lax.iota(jnp.int32, 16)

    def chunk(c, carry):
        b0 = wid * BPW + c * CHB
        r0 = b0 * K
        pltpu.sync_copy(cids.at[pl.ds(b0, CHB)], vidx)
        pltpu.sync_copy(pids.at[pl.ds(b0, CHB)], pidx)
        pltpu.sync_copy(nids.at[pl.ds(r0, 512)], nidx0)
        pltpu.sync_copy(nids.at[pl.ds(r0 + 512, 128)], nidx1)
        c1 = pltpu.async_copy(ctab.at[vidx], vrows, sem)
        c2 = pltpu.async_copy(xtab.at[pidx], prows, sem)
        c3 = pltpu.async_copy(xtab.at[nidx0], nrows.at[pl.ds(0, 512)], sem)
        c4 = pltpu.async_copy(xtab.at[nidx1], nrows.at[pl.ds(512, 128)], sem)
        c1.wait()
        c2.wait()
        c3.wait()
        c4.wait()

        def per_b(b, carry_b):
            v0 = vrows[b, pl.ds(0, 16)]
            v1 = vrows[b, pl.ds(16, 16)]
            v2 = vrows[b, pl.ds(32, 16)]
            v3 = vrows[b, pl.ds(48, 16)]
            acc_a = jnp.zeros((16,), jnp.float32)
            acc_b = jnp.zeros((16,), jnp.float32)
            for k in range(K):
                r = b * K + k
                t = (v0 * nrows[r, pl.ds(0, 16)]
                     + v1 * nrows[r, pl.ds(16, 16)]
                     + v2 * nrows[r, pl.ds(32, 16)]
                     + v3 * nrows[r, pl.ds(48, 16)])
                s = jnp.sum(t)
                if k < 16:
                    acc_a = jnp.where(lanes == k, s, acc_a)
                else:
                    acc_b = jnp.where(lanes == (k - 16), s, acc_b)
            t = (v0 * prows[b, pl.ds(0, 16)] + v1 * prows[b, pl.ds(16, 16)]
                 + v2 * prows[b, pl.ds(32, 16)] + v3 * prows[b, pl.ds(48, 16)])
            acc_b = jnp.where(lanes == (K - 16), jnp.sum(t), acc_b)
            stage[pl.ds(b * SLOTS, 16)] = acc_a
            stage[pl.ds(b * SLOTS + 16, 16)] = acc_b
            return carry_b

        lax.fori_loop(0, CHB, per_b, 0)
        pltpu.sync_copy(stage, s_out.at[pl.ds(b0 * SLOTS, CHB * SLOTS)])
        return carry

    lax.fori_loop(0, NCHK, chunk, 0)


_fused_cache = []


def _fused_kernel():
    if not _fused_cache:
        _fused_cache.append(pl.kernel(
            _fused_body,
            out_type=jax.ShapeDtypeStruct((B * SLOTS,), jnp.float32),
            mesh=plsc.VectorSubcoreMesh(
                core_axis_name="c", subcore_axis_name="s",
                num_cores=NC, num_subcores=NS),
            scratch_types=[
                pltpu.VMEM((CHB,), jnp.int32),
                pltpu.VMEM((CHB,), jnp.int32),
                pltpu.VMEM((512,), jnp.int32),
                pltpu.VMEM((128,), jnp.int32),
                pltpu.VMEM((CHB, DP), jnp.float32),
                pltpu.VMEM((CHB, DP), jnp.float32),
                pltpu.VMEM((CN, DP), jnp.float32),
                pltpu.VMEM((CHB * SLOTS,), jnp.float32),
                pltpu.SemaphoreType.DMA,
            ],
            compiler_params=pltpu.CompilerParams(needs_layout_passes=False),
        ))
    return _fused_cache[0]


def _logsig(x):
    return jnp.minimum(x, 0.0) - jnp.log1p(jnp.exp(-jnp.abs(x)))


def _loss_body(s_ref, o_ref):
    s = s_ref[...]
    col = lax.broadcasted_iota(jnp.int32, (B, SLOTS), 1)
    neg = jnp.where(col < K, -_logsig(-s), 0.0)
    pos = jnp.where(col == K, -_logsig(s), 0.0)
    o_ref[...] = jnp.sum(neg + pos).reshape(1, 1)


_loss = pl.pallas_call(
    _loss_body,
    out_shape=jax.ShapeDtypeStruct((1, 1), jnp.float32),
)


def kernel(center_ids, pos_ids, neg_ids, center_table, context_table):
    cids = center_ids.astype(jnp.int32)
    pids = pos_ids.astype(jnp.int32)
    nids = neg_ids.reshape(-1).astype(jnp.int32)
    ctab = jnp.pad(center_table, ((0, 0), (0, DP - D)))
    xtab = jnp.pad(context_table, ((0, 0), (0, DP - D)))
    scores = _fused_kernel()(cids, pids, nids, ctab, xtab)
    total = _loss(scores.reshape(B, SLOTS))
    return total[0, 0] / B
